# baseline (speedup 1.0000x reference)
import functools

import jax
import jax.numpy as jnp
from jax import lax
from jax.experimental import pallas as pl
from jax.experimental.pallas import tpu as pltpu
from jax.experimental.pallas import tpu_sc as plsc

N = 10000
E = 320000
NP = 10240
CK = 80
NW = 32
C = 125
EP = NW * C * CK
EPS = 1e-5
ROWS_PER_TILE = NP // 16
ZCOPIES = ROWS_PER_TILE // CK

_MESH = plsc.VectorSubcoreMesh(core_axis_name="c", subcore_axis_name="s")


def _make_agg(dh, nchunks, ck, split, nbuf):
    @functools.partial(
        pl.kernel,
        out_type=jax.ShapeDtypeStruct((2, NP, dh), jnp.float32),
        mesh=_MESH,
        scratch_types=[
            pltpu.VMEM((nchunks, ck), jnp.int32),
            pltpu.VMEM((nchunks, ck), jnp.int32),
            pltpu.VMEM((nbuf * ck, dh), jnp.float32),
            pltpu.VMEM_SHARED((NP, dh), jnp.float32),
        ] + [pltpu.SemaphoreType.DMA] * nbuf,
        compiler_params=pltpu.CompilerParams(use_tc_tiling_on_sc=False),
    )
    def agg(z_h, src_h, dst_h, zero_h, out_h, src_v, dst_v, rows_v, acc_sh,
            *gs):
        cid = lax.axis_index("c")
        sid = lax.axis_index("s")
        w = sid if split else cid * 16 + sid
        ztab = z_h.at[cid] if split else z_h
        pltpu.sync_copy(src_h.at[w], src_v)
        pltpu.sync_copy(dst_h.at[w], dst_v)
        buf0 = rows_v.at[pl.ds(0, ck)]
        pltpu.sync_copy(zero_h, buf0)
        for r in range(ROWS_PER_TILE // ck):
            pltpu.sync_copy(buf0, acc_sh.at[pl.ds(sid * ROWS_PER_TILE + r * ck, ck)])
        plsc.subcore_barrier()

        def buf(j):
            return rows_v.at[pl.ds(j * ck, ck)]

        for j in range(nbuf):
            pltpu.async_copy(ztab.at[src_v.at[j]], buf(j), gs[j])

        def step(i, carry):
            base = i * nbuf
            for j in range(nbuf):
                c = base + j
                pltpu.make_async_copy(ztab.at[src_v.at[c]], buf(j), gs[j]).wait()
                pltpu.sync_copy(buf(j), acc_sh.at[dst_v.at[c]], add=True)
                pltpu.async_copy(ztab.at[src_v.at[c + nbuf]], buf(j), gs[j])
            return carry

        lax.fori_loop(0, nchunks // nbuf - 1, step, 0)
        for j in range(nbuf):
            c = nchunks - nbuf + j
            pltpu.make_async_copy(ztab.at[src_v.at[c]], buf(j), gs[j]).wait()
            pltpu.sync_copy(buf(j), acc_sh.at[dst_v.at[c]], add=True)
        plsc.subcore_barrier()
        pltpu.sync_copy(acc_sh.at[pl.ds(sid * ROWS_PER_TILE, ROWS_PER_TILE)],
                        out_h.at[cid, pl.ds(sid * ROWS_PER_TILE, ROWS_PER_TILE)])

    return agg


C2 = EP // 16 // CK
_agg128 = _make_agg(64, C2, CK, True, 5)
_agg64 = _make_agg(32, C2, CK, True, 5)
_agg16 = _make_agg(16, C, CK, False, 5)


@functools.partial(
    pl.kernel,
    out_type=jax.ShapeDtypeStruct((2, NP, 16), jnp.float32),
    mesh=_MESH,
    scratch_types=[
        pltpu.VMEM((C, CK), jnp.int32),
        pltpu.VMEM((CK, 16), jnp.float32),
        pltpu.VMEM_SHARED((NP, 16), jnp.float32),
    ],
    compiler_params=pltpu.CompilerParams(use_tc_tiling_on_sc=False),
)
def _deg(dst_h, zero_h, ones_h, out_h, dst_v, rows_v, acc_sh):
    cid = lax.axis_index("c")
    sid = lax.axis_index("s")
    w = cid * 16 + sid
    pltpu.sync_copy(dst_h.at[w], dst_v)
    pltpu.sync_copy(zero_h, rows_v)
    for r in range(ZCOPIES):
        pltpu.sync_copy(rows_v, acc_sh.at[pl.ds(sid * ROWS_PER_TILE + r * CK, CK)])
    plsc.subcore_barrier()
    pltpu.sync_copy(ones_h, rows_v)

    def step(c, carry):
        pltpu.sync_copy(rows_v, acc_sh.at[dst_v.at[c]], add=True)
        return carry

    lax.fori_loop(0, C, step, 0)
    plsc.subcore_barrier()
    pltpu.sync_copy(acc_sh.at[pl.ds(sid * ROWS_PER_TILE, ROWS_PER_TILE)],
                    out_h.at[cid, pl.ds(sid * ROWS_PER_TILE, ROWS_PER_TILE)])


def _tc_a_body(x_ref, degp_ref, w1_ref, z1_ref, dinv_ref):
    deg = degp_ref[0, :, 0:1] + degp_ref[1, :, 0:1] + 1.0
    dinv = lax.rsqrt(deg)
    dinv_ref[...] = dinv
    z = jnp.dot(x_ref[...], w1_ref[...],
                preferred_element_type=jnp.float32) * dinv
    z1_ref[0] = z[:, :64]
    z1_ref[1] = z[:, 64:]


def _bn_relu(S, g, be):
    mask = lax.broadcasted_iota(jnp.int32, (NP, 1), 0) < N
    Sm = jnp.where(mask, S, 0.0)
    mean = jnp.sum(Sm, axis=0, keepdims=True) * (1.0 / N)
    cent = S - mean
    var = jnp.sum(jnp.where(mask, cent * cent, 0.0), axis=0,
                  keepdims=True) * (1.0 / N)
    return jax.nn.relu(g * cent / jnp.sqrt(var + EPS) + be)


def _tc_b_body(u_ref, z1_ref, dinv_ref, b1_ref, g1_ref, be1_ref, w2_ref,
               x1_ref, z2_ref):
    dinv = dinv_ref[...]
    U = jnp.concatenate([u_ref[0], u_ref[1]], axis=1)
    Z1 = jnp.concatenate([z1_ref[0], z1_ref[1]], axis=1)
    S = dinv * (U + Z1) + b1_ref[...]
    x1 = _bn_relu(S, g1_ref[...], be1_ref[...])
    x1_ref[...] = x1
    z2 = jnp.dot(x1, w2_ref[...], preferred_element_type=jnp.float32) * dinv
    z2_ref[0] = z2[:, :32]
    z2_ref[1] = z2[:, 32:]


def _tc_c_body(u_ref, z2_ref, x1_ref, dinv_ref, b2_ref, g2_ref, be2_ref,
               w3_ref, z3_ref):
    dinv = dinv_ref[...]
    U = jnp.concatenate([u_ref[0], u_ref[1]], axis=1)
    Z2 = jnp.concatenate([z2_ref[0], z2_ref[1]], axis=1)
    S = dinv * (U + Z2) + b2_ref[...]
    x2 = _bn_relu(S, g2_ref[...], be2_ref[...])
    xres = x1_ref[:, :64] + x2
    z3_ref[...] = jnp.dot(xres, w3_ref[...],
                          preferred_element_type=jnp.float32) * dinv


def _tc_d_body(u_ref, z3_ref, dinv_ref, b3_ref, out_ref):
    S = dinv_ref[...] * (u_ref[0] + u_ref[1] + z3_ref[...])
    O = S[:, 0:2] + b3_ref[...]
    m = jnp.max(O, axis=1, keepdims=True)
    e = jnp.exp(O - m)
    lse = jnp.log(e[:, 0:1] + e[:, 1:2]) + m
    out_ref[...] = O - lse


def kernel(x, edge_index, W1, b1, g1, be1, W2, b2, g2, be2, W3, b3):
    srcp = edge_index[0].reshape(NW, C, CK)
    dstp = edge_index[1].reshape(NW, C, CK)
    srcp2 = srcp.reshape(16, C2, CK)
    dstp2 = dstp.reshape(16, C2, CK)
    xp = jnp.pad(x, ((0, NP - N), (0, 0)))
    zeros16 = jnp.zeros((CK, 16), jnp.float32)
    ones16 = jnp.ones((CK, 16), jnp.float32)
    zeros32 = jnp.zeros((CK, 32), jnp.float32)
    zeros64 = jnp.zeros((CK, 64), jnp.float32)
    W3p = jnp.pad(W3, ((0, 0), (0, 14)))

    degp = _deg(dstp, zeros16, ones16)

    z1, dinv = pl.pallas_call(
        _tc_a_body,
        out_shape=[jax.ShapeDtypeStruct((2, NP, 64), jnp.float32),
                   jax.ShapeDtypeStruct((NP, 1), jnp.float32)],
    )(xp, degp, W1)

    u1 = _agg128(z1, srcp2, dstp2, zeros64)

    x1, z2 = pl.pallas_call(
        _tc_b_body,
        out_shape=[jax.ShapeDtypeStruct((NP, 128), jnp.float32),
                   jax.ShapeDtypeStruct((2, NP, 32), jnp.float32)],
    )(u1, z1, dinv, b1.reshape(1, 128), g1.reshape(1, 128),
      be1.reshape(1, 128), W2)

    u2 = _agg64(z2, srcp2, dstp2, zeros32)

    z3 = pl.pallas_call(
        _tc_c_body,
        out_shape=jax.ShapeDtypeStruct((NP, 16), jnp.float32),
    )(u2, z2, x1, dinv, b2.reshape(1, 64), g2.reshape(1, 64),
      be2.reshape(1, 64), W3p)

    u3 = _agg16(z3, srcp, dstp, zeros16)

    out = pl.pallas_call(
        _tc_d_body,
        out_shape=jax.ShapeDtypeStruct((NP, 2), jnp.float32),
    )(u3, z3, dinv, b3.reshape(1, 2))

    return out[:N]


# speedup vs baseline: 1.0582x; 1.0582x over previous
import functools

import jax
import jax.numpy as jnp
from jax import lax
from jax.experimental import pallas as pl
from jax.experimental.pallas import tpu as pltpu
from jax.experimental.pallas import tpu_sc as plsc

N = 10000
E = 320000
NP = 10240
CK = 128
NW = 32
C = 80
EP = NW * C * CK
EPS = 1e-5
ROWS_PER_TILE = NP // 16
ZCOPIES = ROWS_PER_TILE // CK

_MESH = plsc.VectorSubcoreMesh(core_axis_name="c", subcore_axis_name="s")


def _make_agg(dh, nchunks, ck, split, nbuf):
    @functools.partial(
        pl.kernel,
        out_type=jax.ShapeDtypeStruct((2, NP, dh), jnp.float32),
        mesh=_MESH,
        scratch_types=[
            pltpu.VMEM((nchunks, ck), jnp.int32),
            pltpu.VMEM((nchunks, ck), jnp.int32),
            pltpu.VMEM((nbuf * ck, dh), jnp.float32),
            pltpu.VMEM_SHARED((NP, dh), jnp.float32),
        ] + [pltpu.SemaphoreType.DMA] * nbuf,
        compiler_params=pltpu.CompilerParams(use_tc_tiling_on_sc=False),
    )
    def agg(z_h, src_h, dst_h, zero_h, out_h, src_v, dst_v, rows_v, acc_sh,
            *gs):
        cid = lax.axis_index("c")
        sid = lax.axis_index("s")
        w = sid if split else cid * 16 + sid
        ztab = z_h.at[cid] if split else z_h
        pltpu.sync_copy(src_h.at[w], src_v)
        pltpu.sync_copy(dst_h.at[w], dst_v)
        buf0 = rows_v.at[pl.ds(0, ck)]
        pltpu.sync_copy(zero_h, buf0)
        for r in range(ROWS_PER_TILE // ck):
            pltpu.sync_copy(buf0, acc_sh.at[pl.ds(sid * ROWS_PER_TILE + r * ck, ck)])
        plsc.subcore_barrier()

        def buf(j):
            return rows_v.at[pl.ds(j * ck, ck)]

        for j in range(nbuf):
            pltpu.async_copy(ztab.at[src_v.at[j]], buf(j), gs[j])

        def step(i, carry):
            base = i * nbuf
            for j in range(nbuf):
                c = base + j
                pltpu.make_async_copy(ztab.at[src_v.at[c]], buf(j), gs[j]).wait()
                pltpu.sync_copy(buf(j), acc_sh.at[dst_v.at[c]], add=True)
                pltpu.async_copy(ztab.at[src_v.at[c + nbuf]], buf(j), gs[j])
            return carry

        lax.fori_loop(0, nchunks // nbuf - 1, step, 0)
        for j in range(nbuf):
            c = nchunks - nbuf + j
            pltpu.make_async_copy(ztab.at[src_v.at[c]], buf(j), gs[j]).wait()
            pltpu.sync_copy(buf(j), acc_sh.at[dst_v.at[c]], add=True)
        plsc.subcore_barrier()
        pltpu.sync_copy(acc_sh.at[pl.ds(sid * ROWS_PER_TILE, ROWS_PER_TILE)],
                        out_h.at[cid, pl.ds(sid * ROWS_PER_TILE, ROWS_PER_TILE)])

    return agg


C2 = EP // 16 // CK
_agg128 = _make_agg(64, C2, CK, True, 5)
_agg64 = _make_agg(32, C2, CK, True, 8)
_agg16 = _make_agg(16, C, CK, False, 8)


@functools.partial(
    pl.kernel,
    out_type=jax.ShapeDtypeStruct((2, NP, 16), jnp.float32),
    mesh=_MESH,
    scratch_types=[
        pltpu.VMEM((C, CK), jnp.int32),
        pltpu.VMEM((CK, 16), jnp.float32),
        pltpu.VMEM_SHARED((NP, 16), jnp.float32),
    ],
    compiler_params=pltpu.CompilerParams(use_tc_tiling_on_sc=False),
)
def _deg(dst_h, zero_h, ones_h, out_h, dst_v, rows_v, acc_sh):
    cid = lax.axis_index("c")
    sid = lax.axis_index("s")
    w = cid * 16 + sid
    pltpu.sync_copy(dst_h.at[w], dst_v)
    pltpu.sync_copy(zero_h, rows_v)
    for r in range(ZCOPIES):
        pltpu.sync_copy(rows_v, acc_sh.at[pl.ds(sid * ROWS_PER_TILE + r * CK, CK)])
    plsc.subcore_barrier()
    pltpu.sync_copy(ones_h, rows_v)

    def step(c, carry):
        pltpu.sync_copy(rows_v, acc_sh.at[dst_v.at[c]], add=True)
        return carry

    lax.fori_loop(0, C, step, 0)
    plsc.subcore_barrier()
    pltpu.sync_copy(acc_sh.at[pl.ds(sid * ROWS_PER_TILE, ROWS_PER_TILE)],
                    out_h.at[cid, pl.ds(sid * ROWS_PER_TILE, ROWS_PER_TILE)])


def _tc_a_body(x_ref, degp_ref, w1_ref, z1_ref, dinv_ref):
    deg = degp_ref[0, :, 0:1] + degp_ref[1, :, 0:1] + 1.0
    dinv = lax.rsqrt(deg)
    dinv_ref[...] = dinv
    z = jnp.dot(x_ref[...], w1_ref[...],
                preferred_element_type=jnp.float32) * dinv
    z1_ref[0] = z[:, :64]
    z1_ref[1] = z[:, 64:]


def _bn_relu(S, g, be):
    mask = lax.broadcasted_iota(jnp.int32, (NP, 1), 0) < N
    Sm = jnp.where(mask, S, 0.0)
    mean = jnp.sum(Sm, axis=0, keepdims=True) * (1.0 / N)
    cent = S - mean
    var = jnp.sum(jnp.where(mask, cent * cent, 0.0), axis=0,
                  keepdims=True) * (1.0 / N)
    return jax.nn.relu(g * cent / jnp.sqrt(var + EPS) + be)


def _tc_b_body(u_ref, z1_ref, dinv_ref, b1_ref, g1_ref, be1_ref, w2_ref,
               x1_ref, z2_ref):
    dinv = dinv_ref[...]
    U = jnp.concatenate([u_ref[0], u_ref[1]], axis=1)
    Z1 = jnp.concatenate([z1_ref[0], z1_ref[1]], axis=1)
    S = dinv * (U + Z1) + b1_ref[...]
    x1 = _bn_relu(S, g1_ref[...], be1_ref[...])
    x1_ref[...] = x1
    z2 = jnp.dot(x1, w2_ref[...], preferred_element_type=jnp.float32) * dinv
    z2_ref[0] = z2[:, :32]
    z2_ref[1] = z2[:, 32:]


def _tc_c_body(u_ref, z2_ref, x1_ref, dinv_ref, b2_ref, g2_ref, be2_ref,
               w3_ref, z3_ref):
    dinv = dinv_ref[...]
    U = jnp.concatenate([u_ref[0], u_ref[1]], axis=1)
    Z2 = jnp.concatenate([z2_ref[0], z2_ref[1]], axis=1)
    S = dinv * (U + Z2) + b2_ref[...]
    x2 = _bn_relu(S, g2_ref[...], be2_ref[...])
    xres = x1_ref[:, :64] + x2
    z3_ref[...] = jnp.dot(xres, w3_ref[...],
                          preferred_element_type=jnp.float32) * dinv


def _tc_d_body(u_ref, z3_ref, dinv_ref, b3_ref, out_ref):
    S = dinv_ref[...] * (u_ref[0] + u_ref[1] + z3_ref[...])
    O = S[:, 0:2] + b3_ref[...]
    m = jnp.max(O, axis=1, keepdims=True)
    e = jnp.exp(O - m)
    lse = jnp.log(e[:, 0:1] + e[:, 1:2]) + m
    out_ref[...] = O - lse


def kernel(x, edge_index, W1, b1, g1, be1, W2, b2, g2, be2, W3, b3):
    padn = EP - E
    fill = N + (jnp.arange(padn, dtype=jnp.int32) % (NP - N))
    srcp = jnp.concatenate([edge_index[0], fill]).reshape(NW, C, CK)
    dstp = jnp.concatenate([edge_index[1], fill]).reshape(NW, C, CK)
    srcp2 = srcp.reshape(16, C2, CK)
    dstp2 = dstp.reshape(16, C2, CK)
    xp = jnp.pad(x, ((0, NP - N), (0, 0)))
    zeros16 = jnp.zeros((CK, 16), jnp.float32)
    ones16 = jnp.ones((CK, 16), jnp.float32)
    zeros32 = jnp.zeros((CK, 32), jnp.float32)
    zeros64 = jnp.zeros((CK, 64), jnp.float32)
    W3p = jnp.pad(W3, ((0, 0), (0, 14)))

    degp = _deg(dstp, zeros16, ones16)

    z1, dinv = pl.pallas_call(
        _tc_a_body,
        out_shape=[jax.ShapeDtypeStruct((2, NP, 64), jnp.float32),
                   jax.ShapeDtypeStruct((NP, 1), jnp.float32)],
    )(xp, degp, W1)

    u1 = _agg128(z1, srcp2, dstp2, zeros64)

    x1, z2 = pl.pallas_call(
        _tc_b_body,
        out_shape=[jax.ShapeDtypeStruct((NP, 128), jnp.float32),
                   jax.ShapeDtypeStruct((2, NP, 32), jnp.float32)],
    )(u1, z1, dinv, b1.reshape(1, 128), g1.reshape(1, 128),
      be1.reshape(1, 128), W2)

    u2 = _agg64(z2, srcp2, dstp2, zeros32)

    z3 = pl.pallas_call(
        _tc_c_body,
        out_shape=jax.ShapeDtypeStruct((NP, 16), jnp.float32),
    )(u2, z2, x1, dinv, b2.reshape(1, 64), g2.reshape(1, 64),
      be2.reshape(1, 64), W3p)

    u3 = _agg16(z3, srcp, dstp, zeros16)

    out = pl.pallas_call(
        _tc_d_body,
        out_shape=jax.ShapeDtypeStruct((NP, 2), jnp.float32),
    )(u3, z3, dinv, b3.reshape(1, 2))

    return out[:N]
